# Initial kernel scaffold; baseline (speedup 1.0000x reference)
import jax
import jax.numpy as jnp
from jax.experimental import pallas as pl


def kernel(x, edge_index, batch, W1, b1, W2, b2, Wout, bout):
    raise NotImplementedError("write your pallas kernel here")


# speedup vs baseline: 9.1656x; 9.1656x over previous
import functools

import jax
import jax.numpy as jnp
from jax import lax
from jax.experimental import pallas as pl
from jax.experimental.pallas import tpu as pltpu
from jax.experimental.pallas import tpu_sc as plsc

N = 10000
E = 320000
D = 128
G = 64
NC = 2
NS = 16
NW = NC * NS
CH = 128
NCHUNK = 79
EPW = NCHUNK * CH
PAD_N = 10112
RPS = PAD_N // NS
NBLK = 16
BLK = PAD_N // NBLK

_mesh = plsc.VectorSubcoreMesh(core_axis_name="c", subcore_axis_name="s")
_P = jax.lax.Precision.HIGHEST


def _sc_propagate(hs, src3, dst3, zeros_big):
    @functools.partial(
        pl.kernel,
        out_type=jax.ShapeDtypeStruct((NC, PAD_N, D), jnp.float32),
        mesh=_mesh,
        scratch_types=[
            pltpu.VMEM((NCHUNK, CH), jnp.int32),
            pltpu.VMEM((NCHUNK, CH), jnp.int32),
            pltpu.VMEM((CH, D), jnp.float32),
            pltpu.VMEM_SHARED((PAD_N, D), jnp.float32),
        ],
    )
    def k(hs_hbm, src_hbm, dst_hbm, z_hbm, out_hbm, src_v, dst_v, rows_v, acc):
        c = lax.axis_index("c")
        s = lax.axis_index("s")
        base = s * RPS

        @pl.when(c == 0)
        def _():
            pltpu.sync_copy(hs_hbm.at[pl.ds(base, RPS)], acc.at[pl.ds(base, RPS)])

        @pl.when(c != 0)
        def _():
            pltpu.sync_copy(z_hbm.at[pl.ds(base, RPS)], acc.at[pl.ds(base, RPS)])

        pltpu.sync_copy(src_hbm.at[c].at[s], src_v)
        pltpu.sync_copy(dst_hbm.at[c].at[s], dst_v)
        plsc.subcore_barrier()

        @pl.loop(0, NCHUNK)
        def _(j):
            pltpu.sync_copy(hs_hbm.at[src_v.at[j]], rows_v)
            pltpu.sync_copy(rows_v, acc.at[dst_v.at[j]], add=True)

        plsc.subcore_barrier()
        pltpu.sync_copy(acc.at[pl.ds(base, RPS)],
                        out_hbm.at[c].at[pl.ds(base, RPS)])

    return k(hs, src3, dst3, zeros_big)


def _dinv_from_parts(degp_blk):
    deg = degp_blk[0, :, 0:1] + degp_blk[1, :, 0:1]
    return lax.rsqrt(deg)


def _tc_prescale1_body(x_ref, w1_ref, degp_ref, hs_ref):
    h = lax.dot_general(x_ref[...], w1_ref[...], (((1,), (1,)), ((), ())),
                        preferred_element_type=jnp.float32, precision=_P)
    hs_ref[...] = h * _dinv_from_parts(degp_ref[...])


def _tc_prescale1(x_pad, W1, degp):
    return pl.pallas_call(
        _tc_prescale1_body,
        grid=(NBLK,),
        in_specs=[
            pl.BlockSpec((BLK, D), lambda i: (i, 0)),
            pl.BlockSpec((D, D), lambda i: (0, 0)),
            pl.BlockSpec((NC, BLK, D), lambda i: (0, i, 0)),
        ],
        out_specs=pl.BlockSpec((BLK, D), lambda i: (i, 0)),
        out_shape=jax.ShapeDtypeStruct((PAD_N, D), jnp.float32),
    )(x_pad, W1, degp)


def _tc_mid_body(acc_ref, degp_ref, b1_ref, w2_ref, hs2_ref):
    dinv = _dinv_from_parts(degp_ref[...])
    ssum = acc_ref[0] + acc_ref[1]
    out1 = jnp.maximum(ssum * dinv + b1_ref[...], 0.0)
    h2 = lax.dot_general(out1, w2_ref[...], (((1,), (1,)), ((), ())),
                         preferred_element_type=jnp.float32, precision=_P)
    hs2_ref[...] = h2 * dinv


def _tc_mid(acc1, degp, b1, W2):
    return pl.pallas_call(
        _tc_mid_body,
        grid=(NBLK,),
        in_specs=[
            pl.BlockSpec((NC, BLK, D), lambda i: (0, i, 0)),
            pl.BlockSpec((NC, BLK, D), lambda i: (0, i, 0)),
            pl.BlockSpec((1, D), lambda i: (0, 0)),
            pl.BlockSpec((D, D), lambda i: (0, 0)),
        ],
        out_specs=pl.BlockSpec((BLK, D), lambda i: (i, 0)),
        out_shape=jax.ShapeDtypeStruct((PAD_N, D), jnp.float32),
    )(acc1, degp, b1, W2)


def _tc_final_body(acc_ref, degp_ref, b2_ref, batch_ref, wout_ref, bout_ref,
                   out_ref, sums_ref, cnts_ref):
    i = pl.program_id(0)

    @pl.when(i == 0)
    def _():
        sums_ref[...] = jnp.zeros_like(sums_ref)
        cnts_ref[...] = jnp.zeros_like(cnts_ref)

    dinv = _dinv_from_parts(degp_ref[...])
    ssum = acc_ref[0] + acc_ref[1]
    out2 = jnp.maximum(ssum * dinv + b2_ref[...], 0.0)
    rid = i * BLK + lax.broadcasted_iota(jnp.int32, (BLK, 1), 0)
    out2 = jnp.where(rid < N, out2, 0.0)

    seg = batch_ref[0]
    gi = lax.broadcasted_iota(jnp.int32, (G, BLK), 0)
    mask = (gi == seg).astype(jnp.float32)
    sums_ref[...] += lax.dot_general(mask, out2, (((1,), (0,)), ((), ())),
                                     preferred_element_type=jnp.float32,
                                     precision=_P)
    cnts_ref[...] += jnp.broadcast_to(jnp.sum(mask, axis=1, keepdims=True),
                                      cnts_ref.shape)

    @pl.when(i == NBLK - 1)
    def _():
        g = sums_ref[...] / jnp.maximum(cnts_ref[...], 1.0)
        out_ref[...] = (jnp.sum(g * wout_ref[...], axis=1, keepdims=True)
                        + bout_ref[0, 0])


def _tc_final(acc2, degp, b2, batch3, Wout, bout):
    return pl.pallas_call(
        _tc_final_body,
        grid=(NBLK,),
        in_specs=[
            pl.BlockSpec((NC, BLK, D), lambda i: (0, i, 0)),
            pl.BlockSpec((NC, BLK, D), lambda i: (0, i, 0)),
            pl.BlockSpec((1, D), lambda i: (0, 0)),
            pl.BlockSpec((1, 1, BLK), lambda i: (i, 0, 0)),
            pl.BlockSpec((1, D), lambda i: (0, 0)),
            pl.BlockSpec((1, 1), lambda i: (0, 0)),
        ],
        out_specs=pl.BlockSpec((G, 1), lambda i: (0, 0)),
        out_shape=jax.ShapeDtypeStruct((G, 1), jnp.float32),
        scratch_shapes=[
            pltpu.VMEM((G, D), jnp.float32),
            pltpu.VMEM((G, D), jnp.float32),
        ],
    )(acc2, degp, b2, batch3, Wout, bout)


def kernel(x, edge_index, batch, W1, b1, W2, b2, Wout, bout):
    src = edge_index[0].astype(jnp.int32)
    dst = edge_index[1].astype(jnp.int32)
    pad_e = NW * EPW - E
    src3 = jnp.concatenate([src, jnp.zeros((pad_e,), jnp.int32)]
                           ).reshape(NC, NS, NCHUNK, CH)
    dst3 = jnp.concatenate([dst, jnp.full((pad_e,), N, jnp.int32)]
                           ).reshape(NC, NS, NCHUNK, CH)
    batch3 = jnp.concatenate([batch.astype(jnp.int32),
                              jnp.full((PAD_N - N,), G, jnp.int32)]
                             ).reshape(NBLK, 1, BLK)
    x_pad = jnp.zeros((PAD_N, D), jnp.float32).at[:N].set(x)
    zeros_big = jnp.zeros((PAD_N, D), jnp.float32)
    ones_big = jnp.ones((PAD_N, D), jnp.float32)

    degp = _sc_propagate(ones_big, src3, dst3, zeros_big)
    hs1 = _tc_prescale1(x_pad, W1, degp)
    acc1 = _sc_propagate(hs1, src3, dst3, zeros_big)
    hs2 = _tc_mid(acc1, degp, b1.reshape(1, D), W2)
    acc2 = _sc_propagate(hs2, src3, dst3, zeros_big)
    return _tc_final(acc2, degp, b2.reshape(1, D), batch3,
                     Wout.reshape(1, D), bout.reshape(1, 1))
